# baseline (speedup 1.0000x reference)
import functools

import jax
import jax.numpy as jnp
import numpy as np
from jax import lax
from jax.experimental import pallas as pl
from jax.experimental.pallas import tpu as pltpu
from jax.experimental.pallas import tpu_sc as plsc

ALPHA = 0.05
GAMMA = 0.9
C1 = 1.0 - ALPHA
C2 = 1.0 - ALPHA + ALPHA * GAMMA
GOV = ALPHA * GAMMA / C1
AH = 0.5 * ALPHA

L = 16
B = 64
T = 2048
TW = 2 * T
NWORK = 32
EPW = B // NWORK

_LN1 = float(np.log(C1))
_LN2 = float(np.log(C2))


def _presence_body(la_ref, f_ref):
    la = la_ref[...]
    prv = pltpu.roll(la, 1, 1)
    odd = lax.iota(jnp.int32, TW) % 2 == 1
    odb = jnp.broadcast_to(odd[None, :], (B, TW))
    p0 = jnp.any(jnp.logical_and(prv >= la, odb))
    p1 = jnp.any(jnp.logical_and(la > prv, odb))
    lane = lax.iota(jnp.int32, 128)[None, :]
    ones = jnp.ones((8, 128), jnp.float32)
    zero = jnp.zeros((8, 128), jnp.float32)
    f = jnp.where(jnp.logical_and(p0, lane < 16), ones, zero)
    f_ref[...] = f + jnp.where(
        jnp.logical_and(p1, jnp.logical_and(lane >= 16, lane < 32)), ones, zero)


def _lane_bcast(x, idx):
    dn = lax.GatherDimensionNumbers(
        offset_dims=(), collapsed_slice_dims=(0,), start_index_map=(0,))
    return lax.gather(x, idx[:, None], dn, slice_sizes=(1,),
                      mode=lax.GatherScatterMode.PROMISE_IN_BOUNDS,
                      indices_are_sorted=True, unique_indices=False)


def _scan_body(r_hbm, f_hbm, q_hbm,
               ra_v, rb_v, qa_v, qb_v, f_v,
               sema, semb, semqa, semqb):
    cid = lax.axis_index("c")
    sid = lax.axis_index("s")
    wid = sid * 2 + cid
    epa = wid * EPW
    epb = epa + 1

    cpa = pltpu.async_copy(r_hbm.at[pl.ds(epa * TW, TW)], ra_v, sema)
    cpb = pltpu.async_copy(r_hbm.at[pl.ds(epb * TW, TW)], rb_v, semb)

    pltpu.sync_copy(f_hbm.at[0], f_v)
    flag0 = f_v[pl.ds(0, L)] > 0.5
    flag1 = f_v[pl.ds(L, L)] > 0.5

    ki = lax.iota(jnp.int32, L)
    kf = ki.astype(jnp.float32)
    cn1 = jnp.exp(kf * jnp.float32(-_LN1))
    cp1 = jnp.exp(kf * jnp.float32(_LN1))
    cs1 = cp1 * jnp.float32(C1)
    cn2 = jnp.exp(kf * jnp.float32(-_LN2))
    cp2 = jnp.exp(kf * jnp.float32(_LN2))
    cs2 = cp2 * jnp.float32(C2)
    idx15 = ki * 0 + (L - 1)
    ones = jnp.zeros((L,), jnp.float32) + 1.0
    m0 = jnp.where(flag0, 2.0 * AH * ones, AH * ones)
    o0 = jnp.where(flag0, AH * ones, 0.0 * ones)
    m1 = jnp.where(flag1, 2.0 * AH * ones, AH * ones)
    o1 = jnp.where(flag1, AH * ones, 0.0 * ones)

    cpa.wait()
    cpb.wait()

    def chunk_ep(r_v, q_v, iev, iod, dc, sc):
        b0 = plsc.load_gather(r_v, [iev])
        b1 = plsc.load_gather(r_v, [iod])
        h0 = b0 * m0 - o0
        h1 = b1 * m1 - o1
        uh = h0 - h1
        vh = h0 + h1
        dh = plsc.cumsum(uh * cn1) * cp1 + dc * cs1
        wh = vh + GOV * jnp.abs(dh - uh)
        sh = plsc.cumsum(wh * cn2) * cp2 + sc * cs2
        plsc.store_scatter(q_v, [iev], sh + dh)
        plsc.store_scatter(q_v, [iod], sh - dh)
        return _lane_bcast(dh, idx15), _lane_bcast(sh, idx15)

    zeros = jnp.zeros((L,), jnp.float32)
    halves = zeros + 0.5
    iev0 = ki * 2
    iod0 = iev0 + 1

    @plsc.parallel_loop(0, T // L, 1, unroll=4,
                        carry=(zeros, halves, zeros, halves))
    def chunk(j, carry):
        dca, sca, dcb, scb = carry
        base = j * (2 * L)
        iev = base + iev0
        iod = base + iod0
        dca, sca = chunk_ep(ra_v, qa_v, iev, iod, dca, sca)
        dcb, scb = chunk_ep(rb_v, qb_v, iev, iod, dcb, scb)
        return dca, sca, dcb, scb

    pltpu.async_copy(qa_v, q_hbm.at[pl.ds(epa * TW, TW)], semqa)
    cpq = pltpu.async_copy(qb_v, q_hbm.at[pl.ds(epb * TW, TW)], semqb)
    pltpu.make_async_copy(qa_v, q_hbm.at[pl.ds(epa * TW, TW)], semqa).wait()
    cpq.wait()


_sc_scan = functools.partial(
    pl.kernel,
    out_type=jax.ShapeDtypeStruct((B * TW,), jnp.float32),
    mesh=plsc.VectorSubcoreMesh(core_axis_name="c", subcore_axis_name="s",
                                num_cores=2, num_subcores=16),
    scratch_types=[
        pltpu.VMEM((TW,), jnp.float32),
        pltpu.VMEM((TW,), jnp.float32),
        pltpu.VMEM((TW,), jnp.float32),
        pltpu.VMEM((TW,), jnp.float32),
        pltpu.VMEM((128,), jnp.float32),
        pltpu.SemaphoreType.DMA,
        pltpu.SemaphoreType.DMA,
        pltpu.SemaphoreType.DMA,
        pltpu.SemaphoreType.DMA,
    ],
    compiler_params=pltpu.CompilerParams(needs_layout_passes=False),
)(_scan_body)


def kernel(state, last_action, rewards):
    del state
    la_flat = last_action.reshape(B, TW)
    r_flat = rewards.reshape(B * TW)
    f = pl.pallas_call(
        _presence_body,
        out_shape=jax.ShapeDtypeStruct((8, 128), jnp.float32),
    )(la_flat)
    q_flat = _sc_scan(r_flat, f)
    return q_flat.reshape(B, T, 2)


# speedup vs baseline: 6.1138x; 6.1138x over previous
import functools

import jax
import jax.numpy as jnp
import numpy as np
from jax import lax
from jax.experimental import pallas as pl
from jax.experimental.pallas import tpu as pltpu
from jax.experimental.pallas import tpu_sc as plsc

ALPHA = 0.05
GAMMA = 0.9
C1 = 1.0 - ALPHA
C2 = 1.0 - ALPHA + ALPHA * GAMMA
GOV = ALPHA * GAMMA / C1
AH = 0.5 * ALPHA

L = 16
B = 64
T = 2048
TW = 2 * T
NWORK = 32
EPW = B // NWORK

_LN1 = float(np.log(C1))
_LN2 = float(np.log(C2))


def _presence_body(la_ref, f_ref):
    la = la_ref[...]
    prv = pltpu.roll(la, 1, 1)
    odd = lax.iota(jnp.int32, TW) % 2 == 1
    odb = jnp.broadcast_to(odd[None, :], (B, TW))
    p0 = jnp.any(jnp.logical_and(prv >= la, odb))
    p1 = jnp.any(jnp.logical_and(la > prv, odb))
    lane = lax.iota(jnp.int32, 128)[None, :]
    ones = jnp.ones((8, 128), jnp.float32)
    zero = jnp.zeros((8, 128), jnp.float32)
    f = jnp.where(jnp.logical_and(p0, lane < 16), ones, zero)
    f_ref[...] = f + jnp.where(
        jnp.logical_and(p1, jnp.logical_and(lane >= 16, lane < 32)), ones, zero)


def _lane_bcast(x, idx):
    dn = lax.GatherDimensionNumbers(
        offset_dims=(), collapsed_slice_dims=(0,), start_index_map=(0,))
    return lax.gather(x, idx[:, None], dn, slice_sizes=(1,),
                      mode=lax.GatherScatterMode.PROMISE_IN_BOUNDS,
                      indices_are_sorted=True, unique_indices=False)


def _scan_body(r_hbm, f_hbm, q_hbm,
               ra_v, rb_v, qa_v, qb_v, f_v,
               sema, semb, semqa, semqb):
    cid = lax.axis_index("c")
    sid = lax.axis_index("s")
    wid = sid * 2 + cid
    epa = wid * EPW
    epb = epa + 1

    cpa = pltpu.async_copy(r_hbm.at[epa], ra_v, sema)
    cpb = pltpu.async_copy(r_hbm.at[epb], rb_v, semb)

    pltpu.sync_copy(f_hbm.at[0], f_v)
    flag0 = f_v[pl.ds(0, L)] > 0.5
    flag1 = f_v[pl.ds(L, L)] > 0.5

    ki = lax.iota(jnp.int32, L)
    kf = ki.astype(jnp.float32)
    cn1 = jnp.exp(kf * jnp.float32(-_LN1))
    cp1 = jnp.exp(kf * jnp.float32(_LN1))
    cs1 = cp1 * jnp.float32(C1)
    cn2 = jnp.exp(kf * jnp.float32(-_LN2))
    cp2 = jnp.exp(kf * jnp.float32(_LN2))
    cs2 = cp2 * jnp.float32(C2)
    idx15 = ki * 0 + (L - 1)
    ones = jnp.zeros((L,), jnp.float32) + 1.0
    m0 = jnp.where(flag0, 2.0 * AH * ones, AH * ones)
    o0 = jnp.where(flag0, AH * ones, 0.0 * ones)
    m1 = jnp.where(flag1, 2.0 * AH * ones, AH * ones)
    o1 = jnp.where(flag1, AH * ones, 0.0 * ones)

    cpa.wait()
    cpb.wait()

    def chunk_ep(r_v, q_v, iev, iod, dc, sc):
        b0 = plsc.load_gather(r_v, [iev])
        b1 = plsc.load_gather(r_v, [iod])
        h0 = b0 * m0 - o0
        h1 = b1 * m1 - o1
        uh = h0 - h1
        vh = h0 + h1
        dh = plsc.cumsum(uh * cn1) * cp1 + dc * cs1
        wh = vh + GOV * jnp.abs(dh - uh)
        sh = plsc.cumsum(wh * cn2) * cp2 + sc * cs2
        plsc.store_scatter(q_v, [iev], sh + dh)
        plsc.store_scatter(q_v, [iod], sh - dh)
        return _lane_bcast(dh, idx15), _lane_bcast(sh, idx15)

    zeros = jnp.zeros((L,), jnp.float32)
    halves = zeros + 0.5
    iev0 = ki * 2
    iod0 = iev0 + 1

    @plsc.parallel_loop(0, T // L, 1, unroll=2,
                        carry=(zeros, halves, zeros, halves))
    def chunk(j, carry):
        dca, sca, dcb, scb = carry
        base = j * (2 * L)
        iev = base + iev0
        iod = base + iod0
        dca, sca = chunk_ep(ra_v, qa_v, iev, iod, dca, sca)
        dcb, scb = chunk_ep(rb_v, qb_v, iev, iod, dcb, scb)
        return dca, sca, dcb, scb

    pltpu.async_copy(qa_v, q_hbm.at[epa], semqa)
    cpq = pltpu.async_copy(qb_v, q_hbm.at[epb], semqb)
    pltpu.make_async_copy(qa_v, q_hbm.at[epa], semqa).wait()
    cpq.wait()


_sc_scan = functools.partial(
    pl.kernel,
    out_type=jax.ShapeDtypeStruct((B, TW), jnp.float32),
    mesh=plsc.VectorSubcoreMesh(core_axis_name="c", subcore_axis_name="s",
                                num_cores=2, num_subcores=16),
    scratch_types=[
        pltpu.VMEM((TW,), jnp.float32),
        pltpu.VMEM((TW,), jnp.float32),
        pltpu.VMEM((TW,), jnp.float32),
        pltpu.VMEM((TW,), jnp.float32),
        pltpu.VMEM((128,), jnp.float32),
        pltpu.SemaphoreType.DMA,
        pltpu.SemaphoreType.DMA,
        pltpu.SemaphoreType.DMA,
        pltpu.SemaphoreType.DMA,
    ],
    compiler_params=pltpu.CompilerParams(needs_layout_passes=False),
)(_scan_body)


def kernel(state, last_action, rewards):
    del state
    la_flat = last_action.reshape(B, TW)
    r_flat = rewards.reshape(B, TW)
    f = pl.pallas_call(
        _presence_body,
        out_shape=jax.ShapeDtypeStruct((8, 128), jnp.float32),
    )(la_flat)
    q_flat = _sc_scan(r_flat, f)
    return q_flat.reshape(B, T, 2)


# speedup vs baseline: 6.3492x; 1.0385x over previous
import functools

import jax
import jax.numpy as jnp
import numpy as np
from jax import lax
from jax.experimental import pallas as pl
from jax.experimental.pallas import tpu as pltpu
from jax.experimental.pallas import tpu_sc as plsc

ALPHA = 0.05
GAMMA = 0.9
C1 = 1.0 - ALPHA
C2 = 1.0 - ALPHA + ALPHA * GAMMA
GOV = ALPHA * GAMMA / C1
AH = 0.5 * ALPHA

L = 16
B = 64
T = 2048
TW = 2 * T
NWORK = 32
EPW = B // NWORK
PEP = B // 16

_LN1 = float(np.log(C1))
_LN2 = float(np.log(C2))


def _lane_bcast(x, idx):
    dn = lax.GatherDimensionNumbers(
        offset_dims=(), collapsed_slice_dims=(0,), start_index_map=(0,))
    return lax.gather(x, idx[:, None], dn, slice_sizes=(1,),
                      mode=lax.GatherScatterMode.PROMISE_IN_BOUNDS,
                      indices_are_sorted=True, unique_indices=False)


def _body(la_hbm, r_hbm, q_hbm,
          la_v, ra_v, rb_v, qa_v, qb_v, ex_v, ex2_v, sh_mem,
          seml, sema, semb, semqa, semqb):
    cid = lax.axis_index("c")
    sid = lax.axis_index("s")
    wid = sid * 2 + cid
    epa = wid * EPW
    epb = epa + 1

    cpa = pltpu.async_copy(r_hbm.at[epa], ra_v, sema)
    cpb = pltpu.async_copy(r_hbm.at[epb], rb_v, semb)
    pep = sid * PEP
    cpl = []
    for i in range(PEP):
        cpl.append(pltpu.async_copy(
            la_hbm.at[pep + i], la_v.at[pl.ds(i * TW, TW)], seml))

    ki = lax.iota(jnp.int32, L)
    kf = ki.astype(jnp.float32)
    cn1 = jnp.exp(kf * jnp.float32(-_LN1))
    cp1 = jnp.exp(kf * jnp.float32(_LN1))
    cs1 = cp1 * jnp.float32(C1)
    cn2 = jnp.exp(kf * jnp.float32(-_LN2))
    cp2 = jnp.exp(kf * jnp.float32(_LN2))
    cs2 = cp2 * jnp.float32(C2)
    idx15 = ki * 0 + (L - 1)
    iev0 = ki * 2
    iod0 = iev0 + 1

    for cp in cpl:
        cp.wait()

    big = jnp.zeros((L,), jnp.float32) - 3.4e38
    small = jnp.zeros((L,), jnp.float32) + 3.4e38

    @plsc.parallel_loop(0, PEP * (TW // (2 * L)), 1, unroll=4,
                        carry=(big, small))
    def pres(k, carry):
        mx, mn = carry
        base = k * (2 * L)
        e0 = plsc.load_gather(la_v, [base + iev0])
        e1 = plsc.load_gather(la_v, [base + iod0])
        dd = e0 - e1
        return jnp.maximum(mx, dd), jnp.minimum(mn, dd)

    mx, mn = pres
    mm_v = ex_v
    mm_v[pl.ds(0, L)] = mx
    mm_v[pl.ds(L, L)] = mn
    pltpu.sync_copy(mm_v.at[pl.ds(0, 2 * L)], sh_mem.at[sid])
    plsc.subcore_barrier()
    pltpu.sync_copy(sh_mem, ex2_v)
    gmx = jnp.zeros((L,), jnp.float32) - 3.4e38
    gmn = jnp.zeros((L,), jnp.float32) + 3.4e38
    for i in range(16):
        gmx = jnp.maximum(gmx, ex2_v[i, pl.ds(0, L)])
        gmn = jnp.minimum(gmn, ex2_v[i, pl.ds(L, L)])
    p0 = jnp.max(gmx) >= 0.0
    p1 = jnp.min(gmn) < 0.0

    ones = jnp.zeros((L,), jnp.float32) + 1.0
    m0 = jnp.where(p0, 2.0 * AH * ones, AH * ones)
    o0 = jnp.where(p0, AH * ones, 0.0 * ones)
    m1 = jnp.where(p1, 2.0 * AH * ones, AH * ones)
    o1 = jnp.where(p1, AH * ones, 0.0 * ones)

    cpa.wait()
    cpb.wait()

    def chunk_ep(r_v, q_v, iev, iod, dc, sc):
        b0 = plsc.load_gather(r_v, [iev])
        b1 = plsc.load_gather(r_v, [iod])
        h0 = b0 * m0 - o0
        h1 = b1 * m1 - o1
        uh = h0 - h1
        vh = h0 + h1
        dh = plsc.cumsum(uh * cn1) * cp1 + dc * cs1
        wh = vh + GOV * jnp.abs(dh - uh)
        sh = plsc.cumsum(wh * cn2) * cp2 + sc * cs2
        plsc.store_scatter(q_v, [iev], sh + dh)
        plsc.store_scatter(q_v, [iod], sh - dh)
        return _lane_bcast(dh, idx15), _lane_bcast(sh, idx15)

    zeros = jnp.zeros((L,), jnp.float32)
    halves = zeros + 0.5

    @plsc.parallel_loop(0, T // L, 1, unroll=2,
                        carry=(zeros, halves, zeros, halves))
    def chunk(j, carry):
        dca, sca, dcb, scb = carry
        base = j * (2 * L)
        iev = base + iev0
        iod = base + iod0
        dca, sca = chunk_ep(ra_v, qa_v, iev, iod, dca, sca)
        dcb, scb = chunk_ep(rb_v, qb_v, iev, iod, dcb, scb)
        return dca, sca, dcb, scb

    pltpu.async_copy(qa_v, q_hbm.at[epa], semqa)
    cpq = pltpu.async_copy(qb_v, q_hbm.at[epb], semqb)
    pltpu.make_async_copy(qa_v, q_hbm.at[epa], semqa).wait()
    cpq.wait()


_sc_all = functools.partial(
    pl.kernel,
    out_type=jax.ShapeDtypeStruct((B, TW), jnp.float32),
    mesh=plsc.VectorSubcoreMesh(core_axis_name="c", subcore_axis_name="s",
                                num_cores=2, num_subcores=16),
    scratch_types=[
        pltpu.VMEM((PEP * TW,), jnp.float32),
        pltpu.VMEM((TW,), jnp.float32),
        pltpu.VMEM((TW,), jnp.float32),
        pltpu.VMEM((TW,), jnp.float32),
        pltpu.VMEM((TW,), jnp.float32),
        pltpu.VMEM((2 * L,), jnp.float32),
        pltpu.VMEM((16, 2 * L), jnp.float32),
        pltpu.VMEM_SHARED((16, 2 * L), jnp.float32),
        pltpu.SemaphoreType.DMA,
        pltpu.SemaphoreType.DMA,
        pltpu.SemaphoreType.DMA,
        pltpu.SemaphoreType.DMA,
        pltpu.SemaphoreType.DMA,
    ],
    compiler_params=pltpu.CompilerParams(needs_layout_passes=False),
)(_body)


def kernel(state, last_action, rewards):
    del state
    la_flat = last_action.reshape(B, TW)
    r_flat = rewards.reshape(B, TW)
    q_flat = _sc_all(la_flat, r_flat)
    return q_flat.reshape(B, T, 2)
